# baseline (speedup 1.0000x reference)
import functools

import jax
import jax.numpy as jnp
from jax import lax
from jax.experimental import pallas as pl
from jax.experimental.pallas import tpu as pltpu
from jax.experimental.pallas import tpu_sc as plsc

B = 16384
F = 26
V = 100
D = 16
BN_EPS = 1e-3

NC = 2
NS = 16
NW = NC * NS
BPW = B // NW
CB = 64
E = CB * F
NCHUNK = BPW // CB
G = 128
NG = E // G
VREGS_PER_G = G // 16


def _sc_gather(feats_flat, tables_flat):
    mesh = plsc.VectorSubcoreMesh(
        core_axis_name="c", subcore_axis_name="s", num_cores=NC, num_subcores=NS
    )

    @functools.partial(
        pl.kernel,
        out_type=jax.ShapeDtypeStruct((B * F, D), jnp.float32),
        mesh=mesh,
        scratch_types=[
            pltpu.VMEM((E,), jnp.int32),
            pltpu.VMEM((NG, G), jnp.int32),
            pltpu.VMEM((E, D), jnp.float32),
            pltpu.SemaphoreType.DMA,
        ],
        compiler_params=pltpu.CompilerParams(use_tc_tiling_on_sc=False),
    )
    def k(feats_hbm, tables_hbm, out_hbm, fbuf, idxbuf, rows, sem):
        wid = lax.axis_index("s") * NC + lax.axis_index("c")
        wbase = wid * (BPW * F)

        def chunk_body(c, carry):
            base = wbase + c * E
            pltpu.sync_copy(feats_hbm.at[pl.ds(base, E)], fbuf)

            def idx_body(g, carry2):
                for jj in range(VREGS_PER_G):
                    off = g * G + jj * 16
                    v = fbuf[pl.ds(off, 16)]
                    pos = off + lax.iota(jnp.int32, 16)
                    f = lax.rem(pos, F)
                    valid = (v >= 0) & (v < V)
                    m = jnp.where(valid, v + 1, 0) + f * (V + 1)
                    idxbuf[g, pl.ds(jj * 16, 16)] = m
                return carry2

            lax.fori_loop(0, NG, idx_body, 0)

            handles = [
                pltpu.async_copy(
                    tables_hbm.at[idxbuf.at[g]], rows.at[pl.ds(g * G, G)], sem
                )
                for g in range(NG)
            ]
            for h in handles:
                h.wait()
            pltpu.sync_copy(rows, out_hbm.at[pl.ds(base, E)])
            return carry

        lax.fori_loop(0, NCHUNK, chunk_body, 0)

    return k(feats_flat, tables_flat)


BBLK = 1024
H1 = 64
H2 = 32
_DEF = lax.Precision.DEFAULT


def _exact_sel(x, sel_bf16):
    xh = x.astype(jnp.bfloat16)
    xl = (x - xh.astype(jnp.float32)).astype(jnp.bfloat16)
    hi = jnp.dot(xh, sel_bf16, preferred_element_type=jnp.float32)
    lo = jnp.dot(xl, sel_bf16, preferred_element_type=jnp.float32)
    return hi + lo


def _dense_body(emb_ref, mz_ref, me_ref, ms_ref, sw1_ref, sw2_ref,
                w1_ref, b1_ref, bn1_ref, w2_ref, b2_ref, bn2_ref,
                dw_ref, cw_ref, bias_ref, out_ref):
    x = emb_ref[...]
    z = _exact_sel(x, mz_ref[...])
    a = jnp.maximum(jnp.dot(z, sw1_ref[...], precision=_DEF), 0.0)
    a = jnp.maximum(jnp.dot(a, sw2_ref[...], precision=_DEF), 0.0)
    aexp = _exact_sel(a, me_ref[...])
    se = x * aexp
    h = jnp.dot(se, w1_ref[...], precision=_DEF) + b1_ref[...]
    bn1 = bn1_ref[...]
    h = (h - bn1[0:1, :]) / jnp.sqrt(bn1[1:2, :] + BN_EPS) * bn1[2:3, :] + bn1[3:4, :]
    h = jnp.maximum(h, 0.0)
    h = jnp.dot(h, w2_ref[...], precision=_DEF) + b2_ref[...]
    bn2 = bn2_ref[...]
    h = (h - bn2[0:1, :]) / jnp.sqrt(bn2[1:2, :] + BN_EPS) * bn2[2:3, :] + bn2[3:4, :]
    h = jnp.maximum(h, 0.0)
    dnn = jnp.dot(h, dw_ref[...], precision=_DEF)
    s = _exact_sel(se, ms_ref[...])
    ss = _exact_sel(se * se, ms_ref[...])
    cross = 0.5 * (s * s - ss)
    cl = jnp.dot(cross, cw_ref[...], precision=_DEF)
    out_ref[...] = dnn + cl + bias_ref[...]


def _tc_dense(emb, mz, me, ms, sw1, sw2, w1, b1, bn1, w2, b2, bn2,
              dw, cw, bias):
    def const(shape):
        return pl.BlockSpec(shape, lambda i: tuple(0 for _ in shape))

    return pl.pallas_call(
        _dense_body,
        grid=(B // BBLK,),
        in_specs=[
            pl.BlockSpec((BBLK, F * D), lambda i: (i, 0)),
            const((F * D, F)),
            const((F, F * D)),
            const((F * D, D)),
            const((F, sw1.shape[1])),
            const((sw2.shape[0], F)),
            const((F * D, H1)),
            const((1, H1)),
            const((4, H1)),
            const((H1, H2)),
            const((1, H2)),
            const((4, H2)),
            const((H2, 1)),
            const((D, 1)),
            const((1, 1)),
        ],
        out_specs=pl.BlockSpec((BBLK, 1), lambda i: (i, 0)),
        out_shape=jax.ShapeDtypeStruct((B, 1), jnp.float32),
    )(emb, mz, me, ms, sw1, sw2, w1, b1, bn1, w2, b2, bn2, dw, cw, bias)


def kernel(feats, tables, senet_w1, senet_w2, dnn_w1, dnn_b1, bn1_gamma,
           bn1_beta, bn1_mean, bn1_var, dnn_w2, dnn_b2, bn2_gamma, bn2_beta,
           bn2_mean, bn2_var, deep_w, deep_b, cross_w, cross_b):
    emb = _sc_gather(feats.reshape(B * F), tables.reshape(F * (V + 1), D))
    emb = emb.reshape(B, F * D)

    i416 = jnp.arange(F * D)
    mz = ((jnp.arange(F)[None, :] == (i416[:, None] // D))
          .astype(jnp.bfloat16) / jnp.bfloat16(D))
    me = (jnp.arange(F)[:, None] == (i416[None, :] // D)).astype(jnp.bfloat16)
    ms = (jnp.arange(D)[None, :] == (i416[:, None] % D)).astype(jnp.bfloat16)

    bn1 = jnp.stack([bn1_mean, bn1_var, bn1_gamma, bn1_beta])
    bn2 = jnp.stack([bn2_mean, bn2_var, bn2_gamma, bn2_beta])
    bias = (deep_b + cross_b).reshape(1, 1)

    return _tc_dense(emb, mz, me, ms, senet_w1, senet_w2,
                     dnn_w1, dnn_b1[None, :], bn1, dnn_w2, dnn_b2[None, :],
                     bn2, deep_w, cross_w, bias)


# speedup vs baseline: 1.1682x; 1.1682x over previous
import functools

import jax
import jax.numpy as jnp
from jax import lax
from jax.experimental import pallas as pl
from jax.experimental.pallas import tpu as pltpu
from jax.experimental.pallas import tpu_sc as plsc

B = 16384
F = 26
V = 100
D = 16
BN_EPS = 1e-3

NC = 2
NS = 16
NW = NC * NS
BPW = B // NW
CB = 64
E = CB * F
NCHUNK = BPW // CB
G = 128
NG = E // G
VREGS_PER_G = G // 16
NGRP = 4
GR0 = CB * 8
GR3 = CB * 2


def _sc_gather(feats_flat, tables_flat):
    mesh = plsc.VectorSubcoreMesh(
        core_axis_name="c", subcore_axis_name="s", num_cores=NC, num_subcores=NS
    )

    @functools.partial(
        pl.kernel,
        out_type=jax.ShapeDtypeStruct((NGRP * B * 8, D), jnp.float32),
        mesh=mesh,
        scratch_types=[
            pltpu.VMEM((E,), jnp.int32),
            pltpu.VMEM((NG, G), jnp.int32),
            pltpu.VMEM((GR0,), jnp.int32),
            pltpu.VMEM((GR0, D), jnp.float32),
            pltpu.VMEM((GR0, D), jnp.float32),
            pltpu.VMEM((GR0, D), jnp.float32),
            pltpu.VMEM((GR3, D), jnp.float32),
            pltpu.SemaphoreType.DMA,
        ],
        compiler_params=pltpu.CompilerParams(use_tc_tiling_on_sc=False, needs_layout_passes=False),
    )
    def k(feats_hbm, tables_hbm, out_hbm, fbuf, idxbuf, pbuf,
          rows0, rows1, rows2, rows3, sem):
        wid = lax.axis_index("s") * NC + lax.axis_index("c")
        out16 = out_hbm
        rows = [rows0, rows1, rows2, rows3]

        def chunk_body(c, carry):
            b0 = wid * BPW + c * CB
            pltpu.sync_copy(feats_hbm.at[pl.ds(b0 * F, E)], fbuf)

            def idx_body(t, carry2):
                for jj in range(VREGS_PER_G):
                    q = t * G + jj * 16 + lax.iota(jnp.int32, 16)
                    g = lax.shift_right_logical(q, 9)
                    last = g == 3
                    rel = q - g * 512
                    brel = jnp.where(last,
                                     lax.shift_right_logical(rel, 1),
                                     lax.shift_right_logical(rel, 3))
                    fm8 = jnp.where(last, rel & 1, rel & 7)
                    f = g * 8 + fm8
                    pos = brel * F + f
                    v = plsc.load_gather(fbuf, [pos])
                    valid = (v >= 0) & (v < V)
                    m = jnp.where(valid, v + 1, 0) + f * (V + 1)
                    idxbuf[t, pl.ds(jj * 16, 16)] = m
                return carry2

            lax.fori_loop(0, NG, idx_body, 0)

            handles = []
            for t in range(NG):
                dst = rows[min(t // 4, 3)]
                handles.append(pltpu.async_copy(
                    tables_hbm.at[idxbuf.at[t]],
                    dst.at[pl.ds((t % 4) * G if t < 12 else 0, G)], sem))
            for h in handles:
                h.wait()

            wh = []
            for g in range(3):
                wh.append(pltpu.async_copy(
                    rows[g], out16.at[pl.ds((g * B + b0) * 8, GR0)], sem))
            wh.append(pltpu.async_copy(
                rows3, out16.at[pl.ds((3 * B + b0) * 8, GR3)], sem))
            for h in wh:
                h.wait()
            return carry

        lax.fori_loop(0, NCHUNK, chunk_body, 0)

    return k(feats_flat, tables_flat)


BBLK = 1024
H1 = 64
H2 = 32
KP = NGRP * 128
_DEF = lax.Precision.DEFAULT


def _exact_sel(x, sel_bf16):
    xh = x.astype(jnp.bfloat16)
    xl = (x - xh.astype(jnp.float32)).astype(jnp.bfloat16)
    hi = jnp.dot(xh, sel_bf16, preferred_element_type=jnp.float32)
    lo = jnp.dot(xl, sel_bf16, preferred_element_type=jnp.float32)
    return hi + lo


def _dense_body(x0_ref, x1_ref, x2_ref, x3_ref, mz_ref, me_ref, ms_ref,
                sw1_ref, sw2_ref, w1_ref, b1_ref, bn1_ref, w2_ref, b2_ref,
                bn2_ref, dw_ref, cw_ref, bias_ref, out_ref):
    lane = lax.broadcasted_iota(jnp.int32, (BBLK, 128), 1)
    x3 = jnp.where(lane < 32, x3_ref[...], 0.0)
    x = jnp.concatenate([x0_ref[...], x1_ref[...], x2_ref[...], x3], axis=1)
    z = _exact_sel(x, mz_ref[...])
    a = jnp.maximum(jnp.dot(z, sw1_ref[...], precision=_DEF), 0.0)
    a = jnp.maximum(jnp.dot(a, sw2_ref[...], precision=_DEF), 0.0)
    aexp = _exact_sel(a, me_ref[...])
    se = x * aexp
    h = jnp.dot(se, w1_ref[...], precision=_DEF) + b1_ref[...]
    bn1 = bn1_ref[...]
    h = (h - bn1[0:1, :]) / jnp.sqrt(bn1[1:2, :] + BN_EPS) * bn1[2:3, :] + bn1[3:4, :]
    h = jnp.maximum(h, 0.0)
    h = jnp.dot(h, w2_ref[...], precision=_DEF) + b2_ref[...]
    bn2 = bn2_ref[...]
    h = (h - bn2[0:1, :]) / jnp.sqrt(bn2[1:2, :] + BN_EPS) * bn2[2:3, :] + bn2[3:4, :]
    h = jnp.maximum(h, 0.0)
    dnn = jnp.dot(h, dw_ref[...], precision=_DEF)
    s = _exact_sel(se, ms_ref[...])
    ss = _exact_sel(se * se, ms_ref[...])
    cross = 0.5 * (s * s - ss)
    cl = jnp.dot(cross, cw_ref[...], precision=_DEF)
    out_ref[...] = dnn + cl + bias_ref[...]


def _tc_dense(emb4, mz, me, ms, sw1, sw2, w1, b1, bn1, w2, b2, bn2,
              dw, cw, bias):
    def const(shape):
        return pl.BlockSpec(shape, lambda i: tuple(0 for _ in shape))

    def panel(g):
        return pl.BlockSpec((BBLK, 128), lambda i, g=g: (g * (B // BBLK) + i, 0))

    return pl.pallas_call(
        _dense_body,
        grid=(B // BBLK,),
        in_specs=[
            panel(0), panel(1), panel(2), panel(3),
            const((KP, F)),
            const((F, KP)),
            const((KP, D)),
            const((F, sw1.shape[1])),
            const((sw2.shape[0], F)),
            const((KP, H1)),
            const((1, H1)),
            const((4, H1)),
            const((H1, H2)),
            const((1, H2)),
            const((4, H2)),
            const((H2, 1)),
            const((D, 1)),
            const((1, 1)),
        ],
        out_specs=pl.BlockSpec((BBLK, 1), lambda i: (i, 0)),
        out_shape=jax.ShapeDtypeStruct((B, 1), jnp.float32),
    )(emb4, emb4, emb4, emb4, mz, me, ms, sw1, sw2, w1, b1, bn1,
      w2, b2, bn2, dw, cw, bias)


def kernel(feats, tables, senet_w1, senet_w2, dnn_w1, dnn_b1, bn1_gamma,
           bn1_beta, bn1_mean, bn1_var, dnn_w2, dnn_b2, bn2_gamma, bn2_beta,
           bn2_mean, bn2_var, deep_w, deep_b, cross_w, cross_b):
    emb4 = _sc_gather(feats.reshape(B * F), tables.reshape(F * (V + 1), D))
    emb4 = emb4.reshape(NGRP * B, 128)

    r = jnp.arange(KP)
    fr = (r // 128) * 8 + (r % 128) // 16
    dr = r % 16
    validr = fr < F
    col = jnp.clip(fr * D + dr, 0, F * D - 1)

    mz = (jnp.where(validr[:, None], jnp.arange(F)[None, :] == fr[:, None], False)
          .astype(jnp.bfloat16) / jnp.bfloat16(D))
    me = (jnp.where(validr[None, :], jnp.arange(F)[:, None] == fr[None, :], False)
          .astype(jnp.bfloat16))
    ms = (jnp.where(validr[:, None], jnp.arange(D)[None, :] == dr[:, None], False)
          .astype(jnp.bfloat16))
    w1p = jnp.where(validr[:, None], dnn_w1[col], 0.0)

    bn1 = jnp.stack([bn1_mean, bn1_var, bn1_gamma, bn1_beta])
    bn2 = jnp.stack([bn2_mean, bn2_var, bn2_gamma, bn2_beta])
    bias = (deep_b + cross_b).reshape(1, 1)

    return _tc_dense(emb4, mz, me, ms, senet_w1, senet_w2,
                     w1p, dnn_b1[None, :], bn1, dnn_w2, dnn_b2[None, :],
                     bn2, deep_w, cross_w, bias)


# speedup vs baseline: 1.1737x; 1.0047x over previous
import functools

import jax
import jax.numpy as jnp
import numpy as np
from jax import lax
from jax.experimental import pallas as pl
from jax.experimental.pallas import tpu as pltpu
from jax.experimental.pallas import tpu_sc as plsc

B = 16384
F = 26
V = 100
D = 16
BN_EPS = 1e-3

NC = 2
NS = 16
NW = NC * NS
BPW = B // NW
CB = 64
E = CB * F
NCHUNK = BPW // CB
G = 128
NG = E // G
VREGS_PER_G = G // 16
NGRP = 4
GR0 = CB * 8
GR3 = CB * 2


def _sc_gather(feats, tables_flat):
    mesh = plsc.VectorSubcoreMesh(
        core_axis_name="c", subcore_axis_name="s", num_cores=NC, num_subcores=NS
    )

    @functools.partial(
        pl.kernel,
        out_type=jax.ShapeDtypeStruct((NGRP * B * 8, D), jnp.float32),
        mesh=mesh,
        scratch_types=[
            pltpu.VMEM((CB, F), jnp.int32),
            pltpu.VMEM((NG, G), jnp.int32),
            pltpu.VMEM((GR0, D), jnp.float32),
            pltpu.VMEM((GR0, D), jnp.float32),
            pltpu.VMEM((GR0, D), jnp.float32),
            pltpu.VMEM((GR3, D), jnp.float32),
            pltpu.SemaphoreType.DMA,
        ],
        compiler_params=pltpu.CompilerParams(use_tc_tiling_on_sc=False, needs_layout_passes=False),
    )
    def k(feats_hbm, tables_hbm, out_hbm, fbuf, idxbuf,
          rows0, rows1, rows2, rows3, sem):
        wid = lax.axis_index("s") * NC + lax.axis_index("c")
        out16 = out_hbm
        rows = [rows0, rows1, rows2, rows3]

        def chunk_body(c, carry):
            b0 = wid * BPW + c * CB
            pltpu.sync_copy(feats_hbm.at[pl.ds(b0, CB), :], fbuf)

            def idx_body(t, carry2):
                for jj in range(VREGS_PER_G):
                    q = t * G + jj * 16 + lax.iota(jnp.int32, 16)
                    g = lax.shift_right_logical(q, 9)
                    last = g == 3
                    rel = q - g * 512
                    brel = jnp.where(last,
                                     lax.shift_right_logical(rel, 1),
                                     lax.shift_right_logical(rel, 3))
                    fm8 = jnp.where(last, rel & 1, rel & 7)
                    f = g * 8 + fm8
                    v = plsc.load_gather(fbuf, [brel, f])
                    valid = (v >= 0) & (v < V)
                    m = jnp.where(valid, v + 1, 0) + f * (V + 1)
                    idxbuf[t, pl.ds(jj * 16, 16)] = m
                return carry2

            lax.fori_loop(0, NG, idx_body, 0)

            handles = []
            for t in range(NG):
                dst = rows[min(t // 4, 3)]
                handles.append(pltpu.async_copy(
                    tables_hbm.at[idxbuf.at[t]],
                    dst.at[pl.ds((t % 4) * G if t < 12 else 0, G)], sem))
            for h in handles:
                h.wait()

            wh = []
            for g in range(3):
                wh.append(pltpu.async_copy(
                    rows[g], out16.at[pl.ds((g * B + b0) * 8, GR0)], sem))
            wh.append(pltpu.async_copy(
                rows3, out16.at[pl.ds((3 * B + b0) * 8, GR3)], sem))
            for h in wh:
                h.wait()
            return carry

        lax.fori_loop(0, NCHUNK, chunk_body, 0)

    return k(feats, tables_flat)


BBLK = 2048
H1 = 64
H2 = 32
KP = NGRP * 128
_DEF = lax.Precision.DEFAULT


def _exact_sel(x, sel_bf16):
    xh = x.astype(jnp.bfloat16)
    xl = (x - xh.astype(jnp.float32)).astype(jnp.bfloat16)
    hi = jnp.dot(xh, sel_bf16, preferred_element_type=jnp.float32)
    lo = jnp.dot(xl, sel_bf16, preferred_element_type=jnp.float32)
    return hi + lo


def _dense_body(x0_ref, x1_ref, x2_ref, x3_ref, mz_ref, me_ref, ms_ref,
                sw1_ref, sw2_ref, w1_ref, b1_ref, bn1_ref, w2_ref, b2_ref,
                bn2_ref, dw_ref, cw_ref, bias_ref, out_ref):
    lane = lax.broadcasted_iota(jnp.int32, (BBLK, 128), 1)
    x3 = jnp.where(lane < 32, x3_ref[...], 0.0)
    x = jnp.concatenate([x0_ref[...], x1_ref[...], x2_ref[...], x3], axis=1)
    z = _exact_sel(x, mz_ref[...])
    a = jnp.maximum(jnp.dot(z, sw1_ref[...], precision=_DEF), 0.0)
    a = jnp.maximum(jnp.dot(a, sw2_ref[...], precision=_DEF), 0.0)
    aexp = _exact_sel(a, me_ref[...])
    se = x * aexp
    h = jnp.dot(se, w1_ref[...], precision=_DEF) + b1_ref[...]
    bn1 = bn1_ref[...]
    h = (h - bn1[0:1, :]) / jnp.sqrt(bn1[1:2, :] + BN_EPS) * bn1[2:3, :] + bn1[3:4, :]
    h = jnp.maximum(h, 0.0)
    h = jnp.dot(h, w2_ref[...], precision=_DEF) + b2_ref[...]
    bn2 = bn2_ref[...]
    h = (h - bn2[0:1, :]) / jnp.sqrt(bn2[1:2, :] + BN_EPS) * bn2[2:3, :] + bn2[3:4, :]
    h = jnp.maximum(h, 0.0)
    dnn = jnp.dot(h, dw_ref[...], precision=_DEF)
    s = _exact_sel(se, ms_ref[...])
    ss = _exact_sel(se * se, ms_ref[...])
    cross = 0.5 * (s * s - ss)
    cl = jnp.dot(cross, cw_ref[...], precision=_DEF)
    out_ref[...] = dnn + cl + bias_ref[...]


def _tc_dense(emb4, mz, me, ms, sw1, sw2, w1, b1, bn1, w2, b2, bn2,
              dw, cw, bias):
    def const(shape):
        return pl.BlockSpec(shape, lambda i: tuple(0 for _ in shape))

    def panel(g):
        return pl.BlockSpec((BBLK, 128), lambda i, g=g: (g * (B // BBLK) + i, 0))

    return pl.pallas_call(
        _dense_body,
        grid=(B // BBLK,),
        in_specs=[
            panel(0), panel(1), panel(2), panel(3),
            const((KP, F)),
            const((F, KP)),
            const((KP, D)),
            const((F, sw1.shape[1])),
            const((sw2.shape[0], F)),
            const((KP, H1)),
            const((1, H1)),
            const((4, H1)),
            const((H1, H2)),
            const((1, H2)),
            const((4, H2)),
            const((H2, 1)),
            const((D, 1)),
            const((1, 1)),
        ],
        out_specs=pl.BlockSpec((BBLK, 1), lambda i: (i, 0)),
        out_shape=jax.ShapeDtypeStruct((B, 1), jnp.float32),
    )(emb4, emb4, emb4, emb4, mz, me, ms, sw1, sw2, w1, b1, bn1,
      w2, b2, bn2, dw, cw, bias)


def kernel(feats, tables, senet_w1, senet_w2, dnn_w1, dnn_b1, bn1_gamma,
           bn1_beta, bn1_mean, bn1_var, dnn_w2, dnn_b2, bn2_gamma, bn2_beta,
           bn2_mean, bn2_var, deep_w, deep_b, cross_w, cross_b):
    emb4 = _sc_gather(feats, tables.reshape(F * (V + 1), D))
    emb4 = emb4.reshape(NGRP * B, 128)

    r = np.arange(KP)
    fr = (r // 128) * 8 + (r % 128) // 16
    dr = r % 16
    validr = fr < F
    col = np.where(validr, fr * D + dr, 0)

    mz = jnp.asarray((validr[:, None] & (np.arange(F)[None, :] == fr[:, None]))
                     .astype(np.float32) / D, dtype=jnp.bfloat16)
    me = jnp.asarray((validr[None, :] & (np.arange(F)[:, None] == fr[None, :]))
                     .astype(np.float32), dtype=jnp.bfloat16)
    ms = jnp.asarray((validr[:, None] & (np.arange(D)[None, :] == dr[:, None]))
                     .astype(np.float32), dtype=jnp.bfloat16)
    w1p = jnp.where(jnp.asarray(validr[:, None]), dnn_w1[col], 0.0)

    bn1 = jnp.stack([bn1_mean, bn1_var, bn1_gamma, bn1_beta])
    bn2 = jnp.stack([bn2_mean, bn2_var, bn2_gamma, bn2_beta])
    bias = (deep_b + cross_b).reshape(1, 1)

    return _tc_dense(emb4, mz, me, ms, senet_w1, senet_w2,
                     w1p, dnn_b1[None, :], bn1, dnn_w2, dnn_b2[None, :],
                     bn2, deep_w, cross_w, bias)
